# baseline (speedup 1.0000x reference)
import functools

import jax
import jax.numpy as jnp
from jax import lax
from jax.experimental import pallas as pl
from jax.experimental.pallas import tpu as pltpu
from jax.experimental.pallas import tpu_sc as plsc

NC = 2
NS = 16
NW = NC * NS

D = 64
C = 128
NBUF = 4


SL = 128
NSLAB_W = 244


def _transpose_kernel(V: int):
    mesh = plsc.VectorSubcoreMesh(
        core_axis_name="c", subcore_axis_name="s",
        num_cores=NC, num_subcores=NS)
    n_full = V // SL
    rem_full = n_full - NW * NSLAB_W
    tail_off = n_full * SL
    tail_w = V - tail_off

    @functools.partial(
        pl.kernel,
        out_type=jax.ShapeDtypeStruct((V, D), jnp.float32),
        name="embed_table_transpose",
        mesh=mesh,
        scratch_types=[
            pltpu.VMEM((2, D, SL), jnp.float32),
            pltpu.VMEM((2, SL, D), jnp.float32),
        ] + [pltpu.SemaphoreType.DMA] * 4,
        compiler_params=pltpu.CompilerParams(
            use_tc_tiling_on_sc=False, needs_layout_passes=False),
    )
    def k(w_hbm, t_hbm, in_v, out_v, si0, si1, so0, so1):
        wid = lax.axis_index("s") * NC + lax.axis_index("c")
        base = wid * NSLAB_W
        sin = (si0, si1)
        sout = (so0, so1)

        def in_copy(s, b):
            return pltpu.make_async_copy(
                w_hbm.at[:, pl.ds(s * SL, SL)], in_v.at[b], sin[b])

        def out_copy(s, b):
            return pltpu.make_async_copy(
                out_v.at[b], t_hbm.at[pl.ds(s * SL, SL)], sout[b])

        iota = lax.iota(jnp.int32, 16)

        def transpose_rows(b, nrows):
            def rows(j, carry):
                for u in range(4):
                    v = j * 4 + u
                    cols = jnp.full((16,), v, jnp.int32)
                    for g in range(4):
                        vals = plsc.load_gather(
                            in_v.at[b], [g * 16 + iota, cols])
                        out_v[b, v, pl.ds(g * 16, 16)] = vals
                return carry

            lax.fori_loop(0, nrows // 4, rows, 0)

        in_copy(base + 0, 0).start()
        in_copy(base + 1, 1).start()

        def body(p, carry):
            for b in range(2):
                s = base + 2 * p + b
                in_copy(s, b).wait()

                @pl.when(p > 0)
                def _():
                    out_copy(s - 2, b).wait()

                transpose_rows(b, SL)
                out_copy(s, b).start()

                @pl.when(p < NSLAB_W // 2 - 1)
                def _():
                    in_copy(s + 2, b).start()
            return carry

        lax.fori_loop(0, NSLAB_W // 2, body, 0)
        out_copy(base + NSLAB_W - 2, 0).wait()
        out_copy(base + NSLAB_W - 1, 1).wait()

        @pl.when(wid < rem_full)
        def _():
            s = NW * NSLAB_W + wid
            in_copy(s, 0).start()
            in_copy(s, 0).wait()
            transpose_rows(0, SL)
            out_copy(s, 0).start()
            out_copy(s, 0).wait()

        @pl.when(wid == rem_full)
        def _():
            tin = pltpu.make_async_copy(
                w_hbm.at[:, pl.ds(tail_off, tail_w)],
                in_v.at[0, :, pl.ds(0, tail_w)], si0)
            tin.start()
            tin.wait()
            transpose_rows(0, tail_w)
            tout = pltpu.make_async_copy(
                out_v.at[0, pl.ds(0, tail_w)],
                t_hbm.at[pl.ds(tail_off, tail_w)], so0)
            tout.start()
            tout.wait()

    return k


def _gather_kernel(n_chunks: int, n_rows: int):
    mesh = plsc.VectorSubcoreMesh(
        core_axis_name="c", subcore_axis_name="s",
        num_cores=NC, num_subcores=NS)
    b_per_w = n_chunks * C

    @functools.partial(
        pl.kernel,
        out_type=jax.ShapeDtypeStruct((NW * b_per_w, 2 * D), jnp.float32),
        name="embed_row_gather",
        mesh=mesh,
        scratch_types=[
            pltpu.VMEM((n_chunks, C), jnp.int32),
            pltpu.VMEM((NBUF, C, D), jnp.float32),
        ] + [pltpu.SemaphoreType.DMA] * NBUF,
        compiler_params=pltpu.CompilerParams(use_tc_tiling_on_sc=False),
    )
    def k(table_hbm, idx_hbm, out_hbm, idx_v, rows_v, *sems):
        wid = lax.axis_index("s") * NC + lax.axis_index("c")
        base = wid * b_per_w
        pltpu.sync_copy(idx_hbm.at[wid], idx_v)

        def start(g, b):
            pltpu.make_async_copy(
                table_hbm.at[idx_v.at[g]], rows_v.at[b], sems[b]).start()

        def finish(g, b):
            pltpu.make_async_copy(
                table_hbm.at[idx_v.at[g]], rows_v.at[b], sems[b]).wait()
            pltpu.sync_copy(
                rows_v.at[b],
                out_hbm.at[pl.ds(base + g * C, C), pl.ds(0, D)])

        for b in range(NBUF):
            start(b, b)

        def body(g0, carry):
            for b in range(NBUF):
                g = g0 + b
                finish(g, b)
                start(g + NBUF, b)
            return carry

        lax.fori_loop(0, (n_chunks - NBUF) // NBUF, lambda i, c: body(i * NBUF, c), 0)
        for b in range(NBUF):
            finish(n_chunks - NBUF + b, b)

    return k


def kernel(x, W_E):
    B, S = x.shape
    n_tok = B * S
    assert n_tok % (NW * C) == 0
    n_chunks = n_tok // (NW * C)
    V = W_E.shape[1]
    table = _transpose_kernel(V)(W_E)
    idx = x.reshape(NW, n_chunks, C)
    out = _gather_kernel(n_chunks, V)(table, idx)
    return out[:, :D].reshape(B, S, D)


# speedup vs baseline: 5.3404x; 5.3404x over previous
import functools

import jax
import jax.numpy as jnp
from jax import lax
from jax.experimental import pallas as pl
from jax.experimental.pallas import tpu as pltpu
from jax.experimental.pallas import tpu_sc as plsc

NC = 2
NS = 16
NW = NC * NS

D = 64
HW = 32
C = 128
NBUF = 4


def _gather_kernel(n_chunks: int, n_rows: int):
    mesh = plsc.VectorSubcoreMesh(
        core_axis_name="c", subcore_axis_name="s",
        num_cores=NC, num_subcores=NS)
    r_per_w = n_chunks * C

    @functools.partial(
        pl.kernel,
        out_type=jax.ShapeDtypeStruct((NW * r_per_w, HW), jnp.float32),
        name="embed_row_gather",
        mesh=mesh,
        scratch_types=[
            pltpu.VMEM((n_chunks, C), jnp.int32),
            pltpu.VMEM((NBUF, C, HW), jnp.float32),
        ] + [pltpu.SemaphoreType.DMA] * NBUF,
        compiler_params=pltpu.CompilerParams(use_tc_tiling_on_sc=False),
    )
    def k(table_hbm, idx_hbm, out_hbm, idx_v, rows_v, *sems):
        wid = lax.axis_index("s") * NC + lax.axis_index("c")
        base = wid * r_per_w
        pltpu.sync_copy(idx_hbm.at[wid], idx_v)

        def start(g, b):
            pltpu.make_async_copy(
                table_hbm.at[idx_v.at[g]], rows_v.at[b], sems[b]).start()

        def finish(g, b):
            pltpu.make_async_copy(
                table_hbm.at[idx_v.at[g]], rows_v.at[b], sems[b]).wait()
            pltpu.sync_copy(rows_v.at[b], out_hbm.at[pl.ds(base + g * C, C)])

        for b in range(NBUF):
            start(b, b)

        def body(g0, carry):
            for b in range(NBUF):
                g = g0 + b
                finish(g, b)
                start(g + NBUF, b)
            return carry

        lax.fori_loop(0, (n_chunks - NBUF) // NBUF, lambda i, c: body(i * NBUF, c), 0)
        for b in range(NBUF):
            finish(n_chunks - NBUF + b, b)

    return k


def kernel(x, W_E):
    B, S = x.shape
    n_tok = B * S
    n_half = n_tok * 2
    assert n_half % (NW * C) == 0
    n_chunks = n_half // (NW * C)
    V = W_E.shape[1]
    table = W_E.T.reshape(2 * V, HW)
    idx = (2 * x.reshape(-1)[:, None] + jnp.arange(2, dtype=x.dtype)
           ).reshape(NW, n_chunks, C)
    out = _gather_kernel(n_chunks, 2 * V)(table, idx)
    return out.reshape(B, S, D)


# speedup vs baseline: 7.8785x; 1.4753x over previous
import functools

import jax
import jax.numpy as jnp
from jax import lax
from jax.experimental import pallas as pl
from jax.experimental.pallas import tpu as pltpu
from jax.experimental.pallas import tpu_sc as plsc

NC = 2
NS = 16
NW = NC * NS

D = 64
C = 128
NBUF = 4


def _gather_kernel(n_chunks: int):
    mesh = plsc.VectorSubcoreMesh(
        core_axis_name="c", subcore_axis_name="s",
        num_cores=NC, num_subcores=NS)
    b_per_w = n_chunks * C

    @functools.partial(
        pl.kernel,
        out_type=jax.ShapeDtypeStruct((NW * b_per_w, 2 * D), jnp.float32),
        name="embed_row_gather",
        mesh=mesh,
        scratch_types=[
            pltpu.VMEM((n_chunks, C), jnp.int32),
            pltpu.VMEM((NBUF, C, D), jnp.float32),
        ] + [pltpu.SemaphoreType.DMA] * NBUF,
        compiler_params=pltpu.CompilerParams(use_tc_tiling_on_sc=False),
    )
    def k(table_hbm, idx_hbm, out_hbm, idx_v, rows_v, *sems):
        wid = lax.axis_index("s") * NC + lax.axis_index("c")
        base = wid * b_per_w
        pltpu.sync_copy(idx_hbm.at[wid], idx_v)

        def start(g, b):
            pltpu.make_async_copy(
                table_hbm.at[idx_v.at[g]], rows_v.at[b], sems[b]).start()

        def finish(g, b):
            pltpu.make_async_copy(
                table_hbm.at[idx_v.at[g]], rows_v.at[b], sems[b]).wait()
            pltpu.sync_copy(
                rows_v.at[b],
                out_hbm.at[pl.ds(base + g * C, C), pl.ds(0, D)])

        for b in range(NBUF):
            start(b, b)

        def body(g0, carry):
            for b in range(NBUF):
                g = g0 + b
                finish(g, b)
                start(g + NBUF, b)
            return carry

        lax.fori_loop(0, (n_chunks - NBUF) // NBUF, lambda i, c: body(i * NBUF, c), 0)
        for b in range(NBUF):
            finish(n_chunks - NBUF + b, b)

    return k


def kernel(x, W_E):
    B, S = x.shape
    n_tok = B * S
    assert n_tok % (NW * C) == 0
    n_chunks = n_tok // (NW * C)
    V = W_E.shape[1]
    table = jnp.pad(W_E.T, ((0, 0), (0, 2 * D - W_E.shape[0]))).reshape(2 * V, D)
    idx = x.reshape(NW, n_chunks, C) * 2
    out = _gather_kernel(n_chunks)(table, idx)
    return out[:, :D].reshape(B, S, D)
